# baseline (speedup 1.0000x reference)
import functools

import jax
import jax.numpy as jnp
from jax import lax
from jax.experimental import pallas as pl
from jax.experimental.pallas import tpu as pltpu
from jax.experimental.pallas import tpu_sc as plsc

N = 10000
E = 320000
D = 128
G = 128

NC = 2
NS = 16
NW = NC * NS
NPAD = 10240
ROWS = NPAD // NS
EPW = E // NW
C = 125
NCHUNK = EPW // C

RB = NPAD
GRID = NPAD // RB

_MESH = dict(core_axis_name="c", subcore_axis_name="s",
             num_cores=NC, num_subcores=NS)


@functools.cache
def _make_deg_pass():
  @functools.partial(
      pl.kernel,
      out_type=jax.ShapeDtypeStruct((NC, NPAD, 16), jnp.float32),
      mesh=plsc.VectorSubcoreMesh(**_MESH),
      compiler_params=pltpu.CompilerParams(use_tc_tiling_on_sc=False),
      scratch_types=[
          pltpu.VMEM((NCHUNK, C), jnp.int32),
          pltpu.VMEM((C, 16), jnp.float32),
          pltpu.VMEM_SHARED((NPAD, 16), jnp.float32),
          pltpu.SemaphoreType.DMA,
          pltpu.SemaphoreType.DMA,
      ],
  )
  def deg_pass(dst_hbm, ones_hbm, zeros_hbm, out_hbm, dstv, ones_v, acc_sh,
               sem0, sem1):
    sems = (sem0, sem1)
    c = lax.axis_index("c")
    s = lax.axis_index("s")
    pltpu.sync_copy(zeros_hbm.at[pl.ds(s * ROWS, ROWS), :],
                    acc_sh.at[pl.ds(s * ROWS, ROWS), :])
    pltpu.sync_copy(ones_hbm, ones_v)
    w = c * NS + s
    pltpu.sync_copy(dst_hbm.at[w], dstv)
    plsc.subcore_barrier()

    def body(i, carry):
      pltpu.sync_copy(ones_v, acc_sh.at[dstv.at[i]], add=True)
      return carry

    lax.fori_loop(0, NCHUNK, body, 0)
    plsc.subcore_barrier()
    pltpu.sync_copy(acc_sh.at[pl.ds(s * ROWS, ROWS), :],
                    out_hbm.at[c, pl.ds(s * ROWS, ROWS), :])

  return deg_pass


@functools.cache
def _make_edge_pass(F):
  @functools.partial(
      pl.kernel,
      out_type=jax.ShapeDtypeStruct((NC, NPAD, F), jnp.float32),
      mesh=plsc.VectorSubcoreMesh(**_MESH),
      compiler_params=pltpu.CompilerParams(use_tc_tiling_on_sc=False),
      scratch_types=[
          pltpu.VMEM((NCHUNK, C), jnp.int32),
          pltpu.VMEM((NCHUNK, C), jnp.int32),
          pltpu.VMEM((C, F), jnp.float32),
          pltpu.VMEM((C, F), jnp.float32),
          pltpu.VMEM((C, F), jnp.float32),
          pltpu.VMEM((C, F), jnp.float32),
          pltpu.VMEM_SHARED((NPAD, F), jnp.float32),
          pltpu.SemaphoreType.DMA,
          pltpu.SemaphoreType.DMA,
          pltpu.SemaphoreType.DMA,
          pltpu.SemaphoreType.DMA,
          pltpu.SemaphoreType.DMA,
          pltpu.SemaphoreType.DMA,
          pltpu.SemaphoreType.DMA,
          pltpu.SemaphoreType.DMA,
      ],
  )
  def edge_pass(table_hbm, src_hbm, dst_hbm, zeros_hbm, out_hbm,
                srcv, dstv, m0, m1, m2, m3, acc_sh,
                g0, g1, g2, g3, s0, s1, s2, s3):
    msgs = (m0, m1, m2, m3)
    gsem = (g0, g1, g2, g3)
    ssem = (s0, s1, s2, s3)
    c = lax.axis_index("c")
    s = lax.axis_index("s")
    pltpu.sync_copy(zeros_hbm.at[pl.ds(s * ROWS, ROWS), :],
                    acc_sh.at[pl.ds(s * ROWS, ROWS), :])
    w = c * NS + s
    pltpu.sync_copy(src_hbm.at[w], srcv)
    pltpu.sync_copy(dst_hbm.at[w], dstv)
    plsc.subcore_barrier()

    pltpu.async_copy(table_hbm.at[srcv.at[0]], msgs[0], gsem[0])
    pltpu.async_copy(table_hbm.at[srcv.at[1]], msgs[1], gsem[1])
    pltpu.async_copy(table_hbm.at[srcv.at[2]], msgs[2], gsem[2])

    def body(j, carry):
      for b in range(4):
        i = 4 * j + b
        pltpu.make_async_copy(table_hbm.at[srcv.at[i]], msgs[b],
                              gsem[b]).wait()

        @pl.when(i + 3 < NCHUNK)
        def _():
          pltpu.async_copy(table_hbm.at[srcv.at[i + 3]],
                           msgs[(b + 3) % 4], gsem[(b + 3) % 4])

        pltpu.sync_copy(msgs[b], acc_sh.at[dstv.at[i]], add=True)

      return carry

    lax.fori_loop(0, NCHUNK // 4, body, 0)
    plsc.subcore_barrier()
    pltpu.sync_copy(acc_sh.at[pl.ds(s * ROWS, ROWS), :],
                    out_hbm.at[c, pl.ds(s * ROWS, ROWS), :])

  return edge_pass


def _dinv_of(deg_ref):
  degsum = deg_ref[0] + deg_ref[1]
  return lax.rsqrt(degsum[:, 0:1] + 1.0)


def _first_body(x_ref, w_ref, deg_ref, out_ref):
  dinv = _dinv_of(deg_ref)
  h = jnp.dot(x_ref[...], w_ref[...], preferred_element_type=jnp.float32, precision=lax.Precision.HIGHEST)
  out_ref[...] = h * dinv


def _first_layer(x, W0, accdeg):
  return pl.pallas_call(
      _first_body,
      grid=(GRID,),
      in_specs=[
          pl.BlockSpec((RB, D), lambda i: (i, 0)),
          pl.BlockSpec((D, 32), lambda i: (0, 0)),
          pl.BlockSpec((NC, RB, 16), lambda i: (0, i, 0)),
      ],
      out_specs=pl.BlockSpec((RB, 32), lambda i: (i, 0)),
      out_shape=jax.ShapeDtypeStruct((NPAD, 32), jnp.float32),
  )(x, W0, accdeg)


def _mid_body(acc_ref, hhat_ref, deg_ref, b_ref, w_ref, out_ref):
  dinv = _dinv_of(deg_ref)
  xn = jax.nn.relu(dinv * (acc_ref[0] + acc_ref[1] + hhat_ref[...])
                   + b_ref[...])
  h = jnp.dot(xn, w_ref[...], preferred_element_type=jnp.float32, precision=lax.Precision.HIGHEST)
  out_ref[...] = h * dinv


def _mid_layer(F, F2, acc, hhat, accdeg, b, W):
  return pl.pallas_call(
      _mid_body,
      grid=(GRID,),
      in_specs=[
          pl.BlockSpec((NC, RB, F), lambda i: (0, i, 0)),
          pl.BlockSpec((RB, F), lambda i: (i, 0)),
          pl.BlockSpec((NC, RB, 16), lambda i: (0, i, 0)),
          pl.BlockSpec((1, F), lambda i: (0, 0)),
          pl.BlockSpec((F, F2), lambda i: (0, 0)),
      ],
      out_specs=pl.BlockSpec((RB, F2), lambda i: (i, 0)),
      out_shape=jax.ShapeDtypeStruct((NPAD, F2), jnp.float32),
  )(acc, hhat, accdeg, b, W)


def _final_body(acc_ref, hhat_ref, deg_ref, b_ref, batch_ref,
                fw1_ref, fb1_ref, fw2_ref, fb2_ref, out_ref,
                sums_ref, cnt_ref):
  i = pl.program_id(0)

  @pl.when(i == 0)
  def _():
    sums_ref[...] = jnp.zeros_like(sums_ref)
    cnt_ref[...] = jnp.zeros_like(cnt_ref)

  dinv = _dinv_of(deg_ref)
  x3 = jax.nn.relu(dinv * (acc_ref[0] + acc_ref[1] + hhat_ref[...])
                   + b_ref[...])
  seg = lax.broadcasted_iota(jnp.int32, (G, 1), 0)
  onehot_t = (seg == batch_ref[...]).astype(jnp.float32)
  sums_ref[...] += jnp.dot(onehot_t, x3, preferred_element_type=jnp.float32, precision=lax.Precision.HIGHEST)
  cnt_ref[...] += jnp.sum(onehot_t, axis=1, keepdims=True)

  @pl.when(i == GRID - 1)
  def _():
    pooled = sums_ref[...] / jnp.maximum(cnt_ref[...], 1.0)
    z = jax.nn.relu(jnp.dot(pooled, fw1_ref[...],
                            preferred_element_type=jnp.float32, precision=lax.Precision.HIGHEST) + fb1_ref[...])
    z2 = jnp.dot(z, fw2_ref[...],
                 preferred_element_type=jnp.float32, precision=lax.Precision.HIGHEST) + fb2_ref[...]
    out_ref[...] = 1.0 / (1.0 + jnp.exp(-z2))


def _final_layer(acc, hhat, accdeg, b2, batch_p, fcW1, fcb1, fcW2, fcb2):
  return pl.pallas_call(
      _final_body,
      grid=(GRID,),
      in_specs=[
          pl.BlockSpec((NC, RB, 64), lambda i: (0, i, 0)),
          pl.BlockSpec((RB, 64), lambda i: (i, 0)),
          pl.BlockSpec((NC, RB, 16), lambda i: (0, i, 0)),
          pl.BlockSpec((1, 64), lambda i: (0, 0)),
          pl.BlockSpec((1, RB), lambda i: (0, i)),
          pl.BlockSpec((64, 32), lambda i: (0, 0)),
          pl.BlockSpec((1, 32), lambda i: (0, 0)),
          pl.BlockSpec((32, 1), lambda i: (0, 0)),
          pl.BlockSpec((1, 1), lambda i: (0, 0)),
      ],
      out_specs=pl.BlockSpec((G, 1), lambda i: (0, 0)),
      out_shape=jax.ShapeDtypeStruct((G, 1), jnp.float32),
      scratch_shapes=[
          pltpu.VMEM((G, 64), jnp.float32),
          pltpu.VMEM((G, 1), jnp.float32),
      ],
  )(acc, hhat, accdeg, b2, batch_p, fcW1, fcb1, fcW2, fcb2)


def kernel(x, edge_index, batch, W0, b0, W1, b1, W2, b2,
           fcW1, fcb1, fcW2, fcb2):
  xp = jnp.pad(x, ((0, NPAD - N), (0, 0)))
  batch_p = jnp.pad(batch, (0, NPAD - N), constant_values=G).reshape(1, NPAD)
  src = edge_index[0].reshape(NW, NCHUNK, C)
  dst = edge_index[1].reshape(NW, NCHUNK, C)
  ones_c16 = jnp.ones((C, 16), jnp.float32)
  z16 = jnp.zeros((NPAD, 16), jnp.float32)
  z32 = jnp.zeros((NPAD, 32), jnp.float32)
  z48 = jnp.zeros((NPAD, 48), jnp.float32)
  z64 = jnp.zeros((NPAD, 64), jnp.float32)

  accdeg = _make_deg_pass()(dst, ones_c16, z16)
  hhat0 = _first_layer(xp, W0, accdeg)
  acc0 = _make_edge_pass(32)(hhat0, src, dst, z32)
  hhat1 = _mid_layer(32, 48, acc0, hhat0, accdeg, b0.reshape(1, -1), W1)
  acc1 = _make_edge_pass(48)(hhat1, src, dst, z48)
  hhat2 = _mid_layer(48, 64, acc1, hhat1, accdeg, b1.reshape(1, -1), W2)
  acc2 = _make_edge_pass(64)(hhat2, src, dst, z64)
  return _final_layer(acc2, hhat2, accdeg, b2.reshape(1, -1), batch_p,
                      fcW1, fcb1.reshape(1, -1), fcW2, fcb2.reshape(1, -1))


# speedup vs baseline: 1.0159x; 1.0159x over previous
import functools

import jax
import jax.numpy as jnp
from jax import lax
from jax.experimental import pallas as pl
from jax.experimental.pallas import tpu as pltpu
from jax.experimental.pallas import tpu_sc as plsc

N = 10000
E = 320000
D = 128
G = 128

NC = 2
NS = 16
NW = NC * NS
NPAD = 10240
ROWS = NPAD // NS
EPW = E // NW
C = 125
NCHUNK = EPW // C

RB = 1024
GRID = NPAD // RB

_MESH = dict(core_axis_name="c", subcore_axis_name="s",
             num_cores=NC, num_subcores=NS)


@functools.cache
def _make_deg_pass():
  @functools.partial(
      pl.kernel,
      out_type=jax.ShapeDtypeStruct((NC, NPAD, 16), jnp.float32),
      mesh=plsc.VectorSubcoreMesh(**_MESH),
      compiler_params=pltpu.CompilerParams(use_tc_tiling_on_sc=False),
      scratch_types=[
          pltpu.VMEM((NCHUNK, C), jnp.int32),
          pltpu.VMEM((C, 16), jnp.float32),
          pltpu.VMEM_SHARED((NPAD, 16), jnp.float32),
          pltpu.SemaphoreType.DMA,
          pltpu.SemaphoreType.DMA,
      ],
  )
  def deg_pass(dst_hbm, ones_hbm, zeros_hbm, out_hbm, dstv, ones_v, acc_sh,
               sem0, sem1):
    sems = (sem0, sem1)
    c = lax.axis_index("c")
    s = lax.axis_index("s")
    pltpu.sync_copy(zeros_hbm.at[pl.ds(s * ROWS, ROWS), :],
                    acc_sh.at[pl.ds(s * ROWS, ROWS), :])
    pltpu.sync_copy(ones_hbm, ones_v)
    w = c * NS + s
    pltpu.sync_copy(dst_hbm.at[w], dstv)
    plsc.subcore_barrier()

    def body(i, carry):
      pltpu.sync_copy(ones_v, acc_sh.at[dstv.at[i]], add=True)
      return carry

    lax.fori_loop(0, NCHUNK, body, 0)
    plsc.subcore_barrier()
    pltpu.sync_copy(acc_sh.at[pl.ds(s * ROWS, ROWS), :],
                    out_hbm.at[c, pl.ds(s * ROWS, ROWS), :])

  return deg_pass


@functools.cache
def _make_edge_pass(F):
  @functools.partial(
      pl.kernel,
      out_type=jax.ShapeDtypeStruct((NC, NPAD, F), jnp.float32),
      mesh=plsc.VectorSubcoreMesh(**_MESH),
      compiler_params=pltpu.CompilerParams(use_tc_tiling_on_sc=False),
      scratch_types=[
          pltpu.VMEM((NCHUNK, C), jnp.int32),
          pltpu.VMEM((NCHUNK, C), jnp.int32),
          pltpu.VMEM((C, F), jnp.float32),
          pltpu.VMEM((C, F), jnp.float32),
          pltpu.VMEM((C, F), jnp.float32),
          pltpu.VMEM((C, F), jnp.float32),
          pltpu.VMEM_SHARED((NPAD, F), jnp.float32),
          pltpu.SemaphoreType.DMA,
          pltpu.SemaphoreType.DMA,
          pltpu.SemaphoreType.DMA,
          pltpu.SemaphoreType.DMA,
          pltpu.SemaphoreType.DMA,
          pltpu.SemaphoreType.DMA,
          pltpu.SemaphoreType.DMA,
          pltpu.SemaphoreType.DMA,
      ],
  )
  def edge_pass(table_hbm, src_hbm, dst_hbm, zeros_hbm, out_hbm,
                srcv, dstv, m0, m1, m2, m3, acc_sh,
                g0, g1, g2, g3, s0, s1, s2, s3):
    msgs = (m0, m1, m2, m3)
    gsem = (g0, g1, g2, g3)
    ssem = (s0, s1, s2, s3)
    c = lax.axis_index("c")
    s = lax.axis_index("s")
    pltpu.sync_copy(zeros_hbm.at[pl.ds(s * ROWS, ROWS), :],
                    acc_sh.at[pl.ds(s * ROWS, ROWS), :])
    w = c * NS + s
    pltpu.sync_copy(src_hbm.at[w], srcv)
    pltpu.sync_copy(dst_hbm.at[w], dstv)
    plsc.subcore_barrier()

    pltpu.async_copy(table_hbm.at[srcv.at[0]], msgs[0], gsem[0])
    pltpu.async_copy(table_hbm.at[srcv.at[1]], msgs[1], gsem[1])
    pltpu.async_copy(table_hbm.at[srcv.at[2]], msgs[2], gsem[2])

    def body(j, carry):
      for b in range(4):
        i = 4 * j + b
        pltpu.make_async_copy(table_hbm.at[srcv.at[i]], msgs[b],
                              gsem[b]).wait()

        @pl.when(i + 3 < NCHUNK)
        def _():
          pltpu.async_copy(table_hbm.at[srcv.at[i + 3]],
                           msgs[(b + 3) % 4], gsem[(b + 3) % 4])

        pltpu.sync_copy(msgs[b], acc_sh.at[dstv.at[i]], add=True)

      return carry

    lax.fori_loop(0, NCHUNK // 4, body, 0)
    plsc.subcore_barrier()
    pltpu.sync_copy(acc_sh.at[pl.ds(s * ROWS, ROWS), :],
                    out_hbm.at[c, pl.ds(s * ROWS, ROWS), :])

  return edge_pass


def _dinv_of(deg_ref):
  degsum = deg_ref[0] + deg_ref[1]
  return lax.rsqrt(degsum[:, 0:1] + 1.0)


def _mm0_body(x_ref, w_ref, out_ref):
  out_ref[...] = jnp.dot(x_ref[...], w_ref[...],
                         preferred_element_type=jnp.float32,
                         precision=lax.Precision.HIGHEST)


def _mm0(x, W0):
  return pl.pallas_call(
      _mm0_body,
      grid=(GRID,),
      in_specs=[
          pl.BlockSpec((RB, D), lambda i: (i, 0)),
          pl.BlockSpec((D, 32), lambda i: (0, 0)),
      ],
      out_specs=pl.BlockSpec((RB, 32), lambda i: (i, 0)),
      out_shape=jax.ShapeDtypeStruct((NPAD, 32), jnp.float32),
  )(x, W0)


def _scale_body(h_ref, deg_ref, out_ref):
  out_ref[...] = h_ref[...] * _dinv_of(deg_ref)


def _first_layer(x, W0, accdeg):
  h0 = _mm0(x, W0)
  return pl.pallas_call(
      _scale_body,
      grid=(GRID,),
      in_specs=[
          pl.BlockSpec((RB, 32), lambda i: (i, 0)),
          pl.BlockSpec((NC, RB, 16), lambda i: (0, i, 0)),
      ],
      out_specs=pl.BlockSpec((RB, 32), lambda i: (i, 0)),
      out_shape=jax.ShapeDtypeStruct((NPAD, 32), jnp.float32),
  )(h0, accdeg)


def _mid_body(acc_ref, hhat_ref, deg_ref, b_ref, w_ref, out_ref):
  dinv = _dinv_of(deg_ref)
  xn = jax.nn.relu(dinv * (acc_ref[0] + acc_ref[1] + hhat_ref[...])
                   + b_ref[...])
  h = jnp.dot(xn, w_ref[...], preferred_element_type=jnp.float32, precision=lax.Precision.HIGHEST)
  out_ref[...] = h * dinv


def _mid_layer(F, F2, acc, hhat, accdeg, b, W):
  return pl.pallas_call(
      _mid_body,
      grid=(GRID,),
      in_specs=[
          pl.BlockSpec((NC, RB, F), lambda i: (0, i, 0)),
          pl.BlockSpec((RB, F), lambda i: (i, 0)),
          pl.BlockSpec((NC, RB, 16), lambda i: (0, i, 0)),
          pl.BlockSpec((1, F), lambda i: (0, 0)),
          pl.BlockSpec((F, F2), lambda i: (0, 0)),
      ],
      out_specs=pl.BlockSpec((RB, F2), lambda i: (i, 0)),
      out_shape=jax.ShapeDtypeStruct((NPAD, F2), jnp.float32),
  )(acc, hhat, accdeg, b, W)


def _final_body(acc_ref, hhat_ref, deg_ref, b_ref, batch_ref,
                fw1_ref, fb1_ref, fw2_ref, fb2_ref, out_ref,
                sums_ref, cnt_ref):
  i = pl.program_id(0)

  @pl.when(i == 0)
  def _():
    sums_ref[...] = jnp.zeros_like(sums_ref)
    cnt_ref[...] = jnp.zeros_like(cnt_ref)

  dinv = _dinv_of(deg_ref)
  x3 = jax.nn.relu(dinv * (acc_ref[0] + acc_ref[1] + hhat_ref[...])
                   + b_ref[...])
  seg = lax.broadcasted_iota(jnp.int32, (G, 1), 0)
  onehot_t = (seg == batch_ref[...]).astype(jnp.float32)
  sums_ref[...] += jnp.dot(onehot_t, x3, preferred_element_type=jnp.float32, precision=lax.Precision.HIGHEST)
  cnt_ref[...] += jnp.sum(onehot_t, axis=1, keepdims=True)

  @pl.when(i == GRID - 1)
  def _():
    pooled = sums_ref[...] / jnp.maximum(cnt_ref[...], 1.0)
    z = jax.nn.relu(jnp.dot(pooled, fw1_ref[...],
                            preferred_element_type=jnp.float32, precision=lax.Precision.HIGHEST) + fb1_ref[...])
    z2 = jnp.dot(z, fw2_ref[...],
                 preferred_element_type=jnp.float32, precision=lax.Precision.HIGHEST) + fb2_ref[...]
    out_ref[...] = 1.0 / (1.0 + jnp.exp(-z2))


def _final_layer(acc, hhat, accdeg, b2, batch_p, fcW1, fcb1, fcW2, fcb2):
  return pl.pallas_call(
      _final_body,
      grid=(GRID,),
      in_specs=[
          pl.BlockSpec((NC, RB, 64), lambda i: (0, i, 0)),
          pl.BlockSpec((RB, 64), lambda i: (i, 0)),
          pl.BlockSpec((NC, RB, 16), lambda i: (0, i, 0)),
          pl.BlockSpec((1, 64), lambda i: (0, 0)),
          pl.BlockSpec((1, RB), lambda i: (0, i)),
          pl.BlockSpec((64, 32), lambda i: (0, 0)),
          pl.BlockSpec((1, 32), lambda i: (0, 0)),
          pl.BlockSpec((32, 1), lambda i: (0, 0)),
          pl.BlockSpec((1, 1), lambda i: (0, 0)),
      ],
      out_specs=pl.BlockSpec((G, 1), lambda i: (0, 0)),
      out_shape=jax.ShapeDtypeStruct((G, 1), jnp.float32),
      scratch_shapes=[
          pltpu.VMEM((G, 64), jnp.float32),
          pltpu.VMEM((G, 1), jnp.float32),
      ],
  )(acc, hhat, accdeg, b2, batch_p, fcW1, fcb1, fcW2, fcb2)


def kernel(x, edge_index, batch, W0, b0, W1, b1, W2, b2,
           fcW1, fcb1, fcW2, fcb2):
  xp = jnp.pad(x, ((0, NPAD - N), (0, 0)))
  batch_p = jnp.pad(batch, (0, NPAD - N), constant_values=G).reshape(1, NPAD)
  src = edge_index[0].reshape(NW, NCHUNK, C)
  dst = edge_index[1].reshape(NW, NCHUNK, C)
  ones_c16 = jnp.ones((C, 16), jnp.float32)
  z16 = jnp.zeros((NPAD, 16), jnp.float32)
  z32 = jnp.zeros((NPAD, 32), jnp.float32)
  z48 = jnp.zeros((NPAD, 48), jnp.float32)
  z64 = jnp.zeros((NPAD, 64), jnp.float32)

  accdeg = _make_deg_pass()(dst, ones_c16, z16)
  hhat0 = _first_layer(xp, W0, accdeg)
  acc0 = _make_edge_pass(32)(hhat0, src, dst, z32)
  hhat1 = _mid_layer(32, 48, acc0, hhat0, accdeg, b0.reshape(1, -1), W1)
  acc1 = _make_edge_pass(48)(hhat1, src, dst, z48)
  hhat2 = _mid_layer(48, 64, acc1, hhat1, accdeg, b1.reshape(1, -1), W2)
  acc2 = _make_edge_pass(64)(hhat2, src, dst, z64)
  return _final_layer(acc2, hhat2, accdeg, b2.reshape(1, -1), batch_p,
                      fcW1, fcb1.reshape(1, -1), fcW2, fcb2.reshape(1, -1))


# speedup vs baseline: 1.0315x; 1.0154x over previous
import functools

import jax
import jax.numpy as jnp
from jax import lax
from jax.experimental import pallas as pl
from jax.experimental.pallas import tpu as pltpu
from jax.experimental.pallas import tpu_sc as plsc

N = 10000
E = 320000
D = 128
G = 128

NC = 2
NS = 16
NW = NC * NS
NPAD = 10240
ROWS = NPAD // NS
EPW = E // NW
C = 125
NCHUNK = EPW // C

RB = 1024
GRID = NPAD // RB

_MESH = dict(core_axis_name="c", subcore_axis_name="s",
             num_cores=NC, num_subcores=NS)


@functools.cache
def _make_deg_pass():
  @functools.partial(
      pl.kernel,
      out_type=jax.ShapeDtypeStruct((NC, NPAD, 16), jnp.float32),
      mesh=plsc.VectorSubcoreMesh(**_MESH),
      compiler_params=pltpu.CompilerParams(use_tc_tiling_on_sc=False),
      scratch_types=[
          pltpu.VMEM((NCHUNK, C), jnp.int32),
          pltpu.VMEM((C, 16), jnp.float32),
          pltpu.VMEM_SHARED((NPAD, 16), jnp.float32),
          pltpu.SemaphoreType.DMA,
          pltpu.SemaphoreType.DMA,
      ],
  )
  def deg_pass(dst_hbm, ones_hbm, zeros_hbm, out_hbm, dstv, ones_v, acc_sh,
               sem0, sem1):
    sems = (sem0, sem1)
    c = lax.axis_index("c")
    s = lax.axis_index("s")
    w = c * NS + s
    pltpu.async_copy(zeros_hbm.at[pl.ds(s * ROWS, ROWS), :],
                     acc_sh.at[pl.ds(s * ROWS, ROWS), :], sem0)
    pltpu.async_copy(ones_hbm, ones_v, sem1)
    pltpu.async_copy(dst_hbm.at[w], dstv, sem0)
    pltpu.make_async_copy(zeros_hbm.at[pl.ds(s * ROWS, ROWS), :],
                          acc_sh.at[pl.ds(s * ROWS, ROWS), :], sem0).wait()
    pltpu.make_async_copy(ones_hbm, ones_v, sem1).wait()
    pltpu.make_async_copy(dst_hbm.at[w], dstv, sem0).wait()
    plsc.subcore_barrier()

    def body(i, carry):
      pltpu.sync_copy(ones_v, acc_sh.at[dstv.at[i]], add=True)
      return carry

    lax.fori_loop(0, NCHUNK, body, 0)
    plsc.subcore_barrier()
    pltpu.sync_copy(acc_sh.at[pl.ds(s * ROWS, ROWS), :],
                    out_hbm.at[c, pl.ds(s * ROWS, ROWS), :])

  return deg_pass


@functools.cache
def _make_edge_pass(F):
  @functools.partial(
      pl.kernel,
      out_type=jax.ShapeDtypeStruct((NC, NPAD, F), jnp.float32),
      mesh=plsc.VectorSubcoreMesh(**_MESH),
      compiler_params=pltpu.CompilerParams(use_tc_tiling_on_sc=False),
      scratch_types=[
          pltpu.VMEM((NCHUNK, C), jnp.int32),
          pltpu.VMEM((NCHUNK, C), jnp.int32),
          pltpu.VMEM((C, F), jnp.float32),
          pltpu.VMEM((C, F), jnp.float32),
          pltpu.VMEM((C, F), jnp.float32),
          pltpu.VMEM((C, F), jnp.float32),
          pltpu.VMEM_SHARED((NPAD, F), jnp.float32),
          pltpu.SemaphoreType.DMA,
          pltpu.SemaphoreType.DMA,
          pltpu.SemaphoreType.DMA,
          pltpu.SemaphoreType.DMA,
          pltpu.SemaphoreType.DMA,
          pltpu.SemaphoreType.DMA,
          pltpu.SemaphoreType.DMA,
          pltpu.SemaphoreType.DMA,
      ],
  )
  def edge_pass(table_hbm, src_hbm, dst_hbm, zeros_hbm, out_hbm,
                srcv, dstv, m0, m1, m2, m3, acc_sh,
                g0, g1, g2, g3, s0, s1, s2, s3):
    msgs = (m0, m1, m2, m3)
    gsem = (g0, g1, g2, g3)
    ssem = (s0, s1, s2, s3)
    c = lax.axis_index("c")
    s = lax.axis_index("s")
    w = c * NS + s
    pltpu.async_copy(zeros_hbm.at[pl.ds(s * ROWS, ROWS), :],
                     acc_sh.at[pl.ds(s * ROWS, ROWS), :], ssem[0])
    pltpu.async_copy(src_hbm.at[w], srcv, ssem[1])
    pltpu.async_copy(dst_hbm.at[w], dstv, ssem[2])
    pltpu.make_async_copy(zeros_hbm.at[pl.ds(s * ROWS, ROWS), :],
                          acc_sh.at[pl.ds(s * ROWS, ROWS), :], ssem[0]).wait()
    pltpu.make_async_copy(src_hbm.at[w], srcv, ssem[1]).wait()
    pltpu.make_async_copy(dst_hbm.at[w], dstv, ssem[2]).wait()
    plsc.subcore_barrier()

    pltpu.async_copy(table_hbm.at[srcv.at[0]], msgs[0], gsem[0])
    pltpu.async_copy(table_hbm.at[srcv.at[1]], msgs[1], gsem[1])
    pltpu.async_copy(table_hbm.at[srcv.at[2]], msgs[2], gsem[2])

    def body(j, carry):
      for b in range(4):
        i = 4 * j + b
        pltpu.make_async_copy(table_hbm.at[srcv.at[i]], msgs[b],
                              gsem[b]).wait()

        @pl.when(i + 3 < NCHUNK)
        def _():
          pltpu.async_copy(table_hbm.at[srcv.at[i + 3]],
                           msgs[(b + 3) % 4], gsem[(b + 3) % 4])

        pltpu.sync_copy(msgs[b], acc_sh.at[dstv.at[i]], add=True)

      return carry

    lax.fori_loop(0, NCHUNK // 4, body, 0)
    plsc.subcore_barrier()
    pltpu.sync_copy(acc_sh.at[pl.ds(s * ROWS, ROWS), :],
                    out_hbm.at[c, pl.ds(s * ROWS, ROWS), :])

  return edge_pass


def _dinv_of(deg_ref):
  degsum = deg_ref[0] + deg_ref[1]
  return lax.rsqrt(degsum[:, 0:1] + 1.0)


def _mm0_body(x_ref, w_ref, out_ref):
  out_ref[...] = jnp.dot(x_ref[...], w_ref[...],
                         preferred_element_type=jnp.float32,
                         precision=lax.Precision.HIGHEST)


def _mm0(x, W0):
  return pl.pallas_call(
      _mm0_body,
      grid=(GRID,),
      in_specs=[
          pl.BlockSpec((RB, D), lambda i: (i, 0)),
          pl.BlockSpec((D, 32), lambda i: (0, 0)),
      ],
      out_specs=pl.BlockSpec((RB, 32), lambda i: (i, 0)),
      out_shape=jax.ShapeDtypeStruct((NPAD, 32), jnp.float32),
  )(x, W0)


def _scale_body(h_ref, deg_ref, out_ref):
  out_ref[...] = h_ref[...] * _dinv_of(deg_ref)


def _first_layer(x, W0, accdeg):
  h0 = _mm0(x, W0)
  return pl.pallas_call(
      _scale_body,
      grid=(GRID,),
      in_specs=[
          pl.BlockSpec((RB, 32), lambda i: (i, 0)),
          pl.BlockSpec((NC, RB, 16), lambda i: (0, i, 0)),
      ],
      out_specs=pl.BlockSpec((RB, 32), lambda i: (i, 0)),
      out_shape=jax.ShapeDtypeStruct((NPAD, 32), jnp.float32),
  )(h0, accdeg)


def _mid_body(acc_ref, hhat_ref, deg_ref, b_ref, w_ref, out_ref):
  dinv = _dinv_of(deg_ref)
  xn = jax.nn.relu(dinv * (acc_ref[0] + acc_ref[1] + hhat_ref[...])
                   + b_ref[...])
  h = jnp.dot(xn, w_ref[...], preferred_element_type=jnp.float32, precision=lax.Precision.HIGHEST)
  out_ref[...] = h * dinv


def _mid_layer(F, F2, acc, hhat, accdeg, b, W):
  return pl.pallas_call(
      _mid_body,
      grid=(GRID,),
      in_specs=[
          pl.BlockSpec((NC, RB, F), lambda i: (0, i, 0)),
          pl.BlockSpec((RB, F), lambda i: (i, 0)),
          pl.BlockSpec((NC, RB, 16), lambda i: (0, i, 0)),
          pl.BlockSpec((1, F), lambda i: (0, 0)),
          pl.BlockSpec((F, F2), lambda i: (0, 0)),
      ],
      out_specs=pl.BlockSpec((RB, F2), lambda i: (i, 0)),
      out_shape=jax.ShapeDtypeStruct((NPAD, F2), jnp.float32),
  )(acc, hhat, accdeg, b, W)


def _final_body(acc_ref, hhat_ref, deg_ref, b_ref, batch_ref,
                fw1_ref, fb1_ref, fw2_ref, fb2_ref, out_ref,
                sums_ref, cnt_ref):
  i = pl.program_id(0)

  @pl.when(i == 0)
  def _():
    sums_ref[...] = jnp.zeros_like(sums_ref)
    cnt_ref[...] = jnp.zeros_like(cnt_ref)

  dinv = _dinv_of(deg_ref)
  x3 = jax.nn.relu(dinv * (acc_ref[0] + acc_ref[1] + hhat_ref[...])
                   + b_ref[...])
  seg = lax.broadcasted_iota(jnp.int32, (G, 1), 0)
  onehot_t = (seg == batch_ref[...]).astype(jnp.float32)
  sums_ref[...] += jnp.dot(onehot_t, x3, preferred_element_type=jnp.float32, precision=lax.Precision.HIGHEST)
  cnt_ref[...] += jnp.sum(onehot_t, axis=1, keepdims=True)

  @pl.when(i == GRID - 1)
  def _():
    pooled = sums_ref[...] / jnp.maximum(cnt_ref[...], 1.0)
    z = jax.nn.relu(jnp.dot(pooled, fw1_ref[...],
                            preferred_element_type=jnp.float32, precision=lax.Precision.HIGHEST) + fb1_ref[...])
    z2 = jnp.dot(z, fw2_ref[...],
                 preferred_element_type=jnp.float32, precision=lax.Precision.HIGHEST) + fb2_ref[...]
    out_ref[...] = 1.0 / (1.0 + jnp.exp(-z2))


def _final_layer(acc, hhat, accdeg, b2, batch_p, fcW1, fcb1, fcW2, fcb2):
  return pl.pallas_call(
      _final_body,
      grid=(GRID,),
      in_specs=[
          pl.BlockSpec((NC, RB, 64), lambda i: (0, i, 0)),
          pl.BlockSpec((RB, 64), lambda i: (i, 0)),
          pl.BlockSpec((NC, RB, 16), lambda i: (0, i, 0)),
          pl.BlockSpec((1, 64), lambda i: (0, 0)),
          pl.BlockSpec((1, RB), lambda i: (0, i)),
          pl.BlockSpec((64, 32), lambda i: (0, 0)),
          pl.BlockSpec((1, 32), lambda i: (0, 0)),
          pl.BlockSpec((32, 1), lambda i: (0, 0)),
          pl.BlockSpec((1, 1), lambda i: (0, 0)),
      ],
      out_specs=pl.BlockSpec((G, 1), lambda i: (0, 0)),
      out_shape=jax.ShapeDtypeStruct((G, 1), jnp.float32),
      scratch_shapes=[
          pltpu.VMEM((G, 64), jnp.float32),
          pltpu.VMEM((G, 1), jnp.float32),
      ],
  )(acc, hhat, accdeg, b2, batch_p, fcW1, fcb1, fcW2, fcb2)


def kernel(x, edge_index, batch, W0, b0, W1, b1, W2, b2,
           fcW1, fcb1, fcW2, fcb2):
  xp = jnp.pad(x, ((0, NPAD - N), (0, 0)))
  batch_p = jnp.pad(batch, (0, NPAD - N), constant_values=G).reshape(1, NPAD)
  src = edge_index[0].reshape(NW, NCHUNK, C)
  dst = edge_index[1].reshape(NW, NCHUNK, C)
  ones_c16 = jnp.ones((C, 16), jnp.float32)
  z16 = jnp.zeros((NPAD, 16), jnp.float32)
  z32 = jnp.zeros((NPAD, 32), jnp.float32)
  z48 = jnp.zeros((NPAD, 48), jnp.float32)
  z64 = jnp.zeros((NPAD, 64), jnp.float32)

  accdeg = _make_deg_pass()(dst, ones_c16, z16)
  hhat0 = _first_layer(xp, W0, accdeg)
  acc0 = _make_edge_pass(32)(hhat0, src, dst, z32)
  hhat1 = _mid_layer(32, 48, acc0, hhat0, accdeg, b0.reshape(1, -1), W1)
  acc1 = _make_edge_pass(48)(hhat1, src, dst, z48)
  hhat2 = _mid_layer(48, 64, acc1, hhat1, accdeg, b1.reshape(1, -1), W2)
  acc2 = _make_edge_pass(64)(hhat2, src, dst, z64)
  return _final_layer(acc2, hhat2, accdeg, b2.reshape(1, -1), batch_p,
                      fcW1, fcb1.reshape(1, -1), fcW2, fcb2.reshape(1, -1))


# speedup vs baseline: 1.0662x; 1.0336x over previous
import functools

import jax
import jax.numpy as jnp
from jax import lax
from jax.experimental import pallas as pl
from jax.experimental.pallas import tpu as pltpu
from jax.experimental.pallas import tpu_sc as plsc

N = 10000
E = 320000
D = 128
G = 128

NC = 2
NS = 16
NW = NC * NS
NPAD = 10240
ROWS = NPAD // NS
EPW = E // NW
C = 125
NCHUNK = EPW // C

RB = 2048
GRID = NPAD // RB

_MESH = dict(core_axis_name="c", subcore_axis_name="s",
             num_cores=NC, num_subcores=NS)


@functools.cache
def _make_deg_pass():
  @functools.partial(
      pl.kernel,
      out_type=jax.ShapeDtypeStruct((NC, NPAD, 16), jnp.float32),
      mesh=plsc.VectorSubcoreMesh(**_MESH),
      compiler_params=pltpu.CompilerParams(use_tc_tiling_on_sc=False),
      scratch_types=[
          pltpu.VMEM((NCHUNK, C), jnp.int32),
          pltpu.VMEM((C, 16), jnp.float32),
          pltpu.VMEM_SHARED((NPAD, 16), jnp.float32),
          pltpu.SemaphoreType.DMA,
          pltpu.SemaphoreType.DMA,
      ],
  )
  def deg_pass(dst_hbm, ones_hbm, zeros_hbm, out_hbm, dstv, ones_v, acc_sh,
               sem0, sem1):
    sems = (sem0, sem1)
    c = lax.axis_index("c")
    s = lax.axis_index("s")
    w = c * NS + s
    pltpu.async_copy(zeros_hbm.at[pl.ds(s * ROWS, ROWS), :],
                     acc_sh.at[pl.ds(s * ROWS, ROWS), :], sem0)
    pltpu.async_copy(ones_hbm, ones_v, sem1)
    pltpu.async_copy(dst_hbm.at[w], dstv, sem0)
    pltpu.make_async_copy(zeros_hbm.at[pl.ds(s * ROWS, ROWS), :],
                          acc_sh.at[pl.ds(s * ROWS, ROWS), :], sem0).wait()
    pltpu.make_async_copy(ones_hbm, ones_v, sem1).wait()
    pltpu.make_async_copy(dst_hbm.at[w], dstv, sem0).wait()
    plsc.subcore_barrier()

    def body(i, carry):
      pltpu.sync_copy(ones_v, acc_sh.at[dstv.at[i]], add=True)
      return carry

    lax.fori_loop(0, NCHUNK, body, 0)
    plsc.subcore_barrier()
    pltpu.sync_copy(acc_sh.at[pl.ds(s * ROWS, ROWS), :],
                    out_hbm.at[c, pl.ds(s * ROWS, ROWS), :])

  return deg_pass


@functools.cache
def _make_edge_pass(F):
  @functools.partial(
      pl.kernel,
      out_type=jax.ShapeDtypeStruct((NC, NPAD, F), jnp.float32),
      mesh=plsc.VectorSubcoreMesh(**_MESH),
      compiler_params=pltpu.CompilerParams(use_tc_tiling_on_sc=False),
      scratch_types=[
          pltpu.VMEM((NCHUNK, C), jnp.int32),
          pltpu.VMEM((NCHUNK, C), jnp.int32),
          pltpu.VMEM((C, F), jnp.float32),
          pltpu.VMEM((C, F), jnp.float32),
          pltpu.VMEM((C, F), jnp.float32),
          pltpu.VMEM((C, F), jnp.float32),
          pltpu.VMEM_SHARED((NPAD, F), jnp.float32),
          pltpu.SemaphoreType.DMA,
          pltpu.SemaphoreType.DMA,
          pltpu.SemaphoreType.DMA,
          pltpu.SemaphoreType.DMA,
          pltpu.SemaphoreType.DMA,
          pltpu.SemaphoreType.DMA,
          pltpu.SemaphoreType.DMA,
          pltpu.SemaphoreType.DMA,
      ],
  )
  def edge_pass(table_hbm, src_hbm, dst_hbm, zeros_hbm, out_hbm,
                srcv, dstv, m0, m1, m2, m3, acc_sh,
                g0, g1, g2, g3, s0, s1, s2, s3):
    msgs = (m0, m1, m2, m3)
    gsem = (g0, g1, g2, g3)
    ssem = (s0, s1, s2, s3)
    c = lax.axis_index("c")
    s = lax.axis_index("s")
    w = c * NS + s
    pltpu.async_copy(zeros_hbm.at[pl.ds(s * ROWS, ROWS), :],
                     acc_sh.at[pl.ds(s * ROWS, ROWS), :], ssem[0])
    pltpu.async_copy(src_hbm.at[w], srcv, ssem[1])
    pltpu.async_copy(dst_hbm.at[w], dstv, ssem[2])
    pltpu.make_async_copy(zeros_hbm.at[pl.ds(s * ROWS, ROWS), :],
                          acc_sh.at[pl.ds(s * ROWS, ROWS), :], ssem[0]).wait()
    pltpu.make_async_copy(src_hbm.at[w], srcv, ssem[1]).wait()
    pltpu.make_async_copy(dst_hbm.at[w], dstv, ssem[2]).wait()
    plsc.subcore_barrier()

    pltpu.async_copy(table_hbm.at[srcv.at[0]], msgs[0], gsem[0])
    pltpu.async_copy(table_hbm.at[srcv.at[1]], msgs[1], gsem[1])
    pltpu.async_copy(table_hbm.at[srcv.at[2]], msgs[2], gsem[2])

    def body(j, carry):
      for b in range(4):
        i = 4 * j + b
        pltpu.make_async_copy(table_hbm.at[srcv.at[i]], msgs[b],
                              gsem[b]).wait()

        @pl.when(i + 3 < NCHUNK)
        def _():
          pltpu.async_copy(table_hbm.at[srcv.at[i + 3]],
                           msgs[(b + 3) % 4], gsem[(b + 3) % 4])

        pltpu.sync_copy(msgs[b], acc_sh.at[dstv.at[i]], add=True)

      return carry

    lax.fori_loop(0, NCHUNK // 4, body, 0)
    plsc.subcore_barrier()
    pltpu.sync_copy(acc_sh.at[pl.ds(s * ROWS, ROWS), :],
                    out_hbm.at[c, pl.ds(s * ROWS, ROWS), :])

  return edge_pass


def _dinv_of(deg_ref):
  degsum = deg_ref[0] + deg_ref[1]
  return lax.rsqrt(degsum[:, 0:1] + 1.0)


def _mm0_body(x_ref, w_ref, out_ref):
  out_ref[...] = jnp.dot(x_ref[...], w_ref[...],
                         preferred_element_type=jnp.float32,
                         precision=lax.Precision.HIGHEST)


def _mm0(x, W0):
  return pl.pallas_call(
      _mm0_body,
      grid=(GRID,),
      in_specs=[
          pl.BlockSpec((RB, D), lambda i: (i, 0)),
          pl.BlockSpec((D, 32), lambda i: (0, 0)),
      ],
      out_specs=pl.BlockSpec((RB, 32), lambda i: (i, 0)),
      out_shape=jax.ShapeDtypeStruct((NPAD, 32), jnp.float32),
  )(x, W0)


def _scale_body(h_ref, deg_ref, out_ref):
  out_ref[...] = h_ref[...] * _dinv_of(deg_ref)


def _first_layer(x, W0, accdeg):
  h0 = _mm0(x, W0)
  return pl.pallas_call(
      _scale_body,
      grid=(GRID,),
      in_specs=[
          pl.BlockSpec((RB, 32), lambda i: (i, 0)),
          pl.BlockSpec((NC, RB, 16), lambda i: (0, i, 0)),
      ],
      out_specs=pl.BlockSpec((RB, 32), lambda i: (i, 0)),
      out_shape=jax.ShapeDtypeStruct((NPAD, 32), jnp.float32),
  )(h0, accdeg)


def _mid_body(acc_ref, hhat_ref, deg_ref, b_ref, w_ref, out_ref):
  dinv = _dinv_of(deg_ref)
  xn = jax.nn.relu(dinv * (acc_ref[0] + acc_ref[1] + hhat_ref[...])
                   + b_ref[...])
  h = jnp.dot(xn, w_ref[...], preferred_element_type=jnp.float32, precision=lax.Precision.HIGHEST)
  out_ref[...] = h * dinv


def _mid_layer(F, F2, acc, hhat, accdeg, b, W):
  return pl.pallas_call(
      _mid_body,
      grid=(GRID,),
      in_specs=[
          pl.BlockSpec((NC, RB, F), lambda i: (0, i, 0)),
          pl.BlockSpec((RB, F), lambda i: (i, 0)),
          pl.BlockSpec((NC, RB, 16), lambda i: (0, i, 0)),
          pl.BlockSpec((1, F), lambda i: (0, 0)),
          pl.BlockSpec((F, F2), lambda i: (0, 0)),
      ],
      out_specs=pl.BlockSpec((RB, F2), lambda i: (i, 0)),
      out_shape=jax.ShapeDtypeStruct((NPAD, F2), jnp.float32),
  )(acc, hhat, accdeg, b, W)


def _final_body(acc_ref, hhat_ref, deg_ref, b_ref, batch_ref,
                fw1_ref, fb1_ref, fw2_ref, fb2_ref, out_ref,
                sums_ref, cnt_ref):
  i = pl.program_id(0)

  @pl.when(i == 0)
  def _():
    sums_ref[...] = jnp.zeros_like(sums_ref)
    cnt_ref[...] = jnp.zeros_like(cnt_ref)

  dinv = _dinv_of(deg_ref)
  x3 = jax.nn.relu(dinv * (acc_ref[0] + acc_ref[1] + hhat_ref[...])
                   + b_ref[...])
  seg = lax.broadcasted_iota(jnp.int32, (G, 1), 0)
  onehot_t = (seg == batch_ref[...]).astype(jnp.float32)
  sums_ref[...] += jnp.dot(onehot_t, x3, preferred_element_type=jnp.float32, precision=lax.Precision.HIGHEST)
  cnt_ref[...] += jnp.sum(onehot_t, axis=1, keepdims=True)

  @pl.when(i == GRID - 1)
  def _():
    pooled = sums_ref[...] / jnp.maximum(cnt_ref[...], 1.0)
    z = jax.nn.relu(jnp.dot(pooled, fw1_ref[...],
                            preferred_element_type=jnp.float32, precision=lax.Precision.HIGHEST) + fb1_ref[...])
    z2 = jnp.dot(z, fw2_ref[...],
                 preferred_element_type=jnp.float32, precision=lax.Precision.HIGHEST) + fb2_ref[...]
    out_ref[...] = 1.0 / (1.0 + jnp.exp(-z2))


def _final_layer(acc, hhat, accdeg, b2, batch_p, fcW1, fcb1, fcW2, fcb2):
  return pl.pallas_call(
      _final_body,
      grid=(GRID,),
      in_specs=[
          pl.BlockSpec((NC, RB, 64), lambda i: (0, i, 0)),
          pl.BlockSpec((RB, 64), lambda i: (i, 0)),
          pl.BlockSpec((NC, RB, 16), lambda i: (0, i, 0)),
          pl.BlockSpec((1, 64), lambda i: (0, 0)),
          pl.BlockSpec((1, RB), lambda i: (0, i)),
          pl.BlockSpec((64, 32), lambda i: (0, 0)),
          pl.BlockSpec((1, 32), lambda i: (0, 0)),
          pl.BlockSpec((32, 1), lambda i: (0, 0)),
          pl.BlockSpec((1, 1), lambda i: (0, 0)),
      ],
      out_specs=pl.BlockSpec((G, 1), lambda i: (0, 0)),
      out_shape=jax.ShapeDtypeStruct((G, 1), jnp.float32),
      scratch_shapes=[
          pltpu.VMEM((G, 64), jnp.float32),
          pltpu.VMEM((G, 1), jnp.float32),
      ],
  )(acc, hhat, accdeg, b2, batch_p, fcW1, fcb1, fcW2, fcb2)


def kernel(x, edge_index, batch, W0, b0, W1, b1, W2, b2,
           fcW1, fcb1, fcW2, fcb2):
  xp = jnp.pad(x, ((0, NPAD - N), (0, 0)))
  batch_p = jnp.pad(batch, (0, NPAD - N), constant_values=G).reshape(1, NPAD)
  src = edge_index[0].reshape(NW, NCHUNK, C)
  dst = edge_index[1].reshape(NW, NCHUNK, C)
  ones_c16 = jnp.ones((C, 16), jnp.float32)
  z16 = jnp.zeros((NPAD, 16), jnp.float32)
  z32 = jnp.zeros((NPAD, 32), jnp.float32)
  z48 = jnp.zeros((NPAD, 48), jnp.float32)
  z64 = jnp.zeros((NPAD, 64), jnp.float32)

  accdeg = _make_deg_pass()(dst, ones_c16, z16)
  hhat0 = _first_layer(xp, W0, accdeg)
  acc0 = _make_edge_pass(32)(hhat0, src, dst, z32)
  hhat1 = _mid_layer(32, 48, acc0, hhat0, accdeg, b0.reshape(1, -1), W1)
  acc1 = _make_edge_pass(48)(hhat1, src, dst, z48)
  hhat2 = _mid_layer(48, 64, acc1, hhat1, accdeg, b1.reshape(1, -1), W2)
  acc2 = _make_edge_pass(64)(hhat2, src, dst, z64)
  return _final_layer(acc2, hhat2, accdeg, b2.reshape(1, -1), batch_p,
                      fcW1, fcb1.reshape(1, -1), fcW2, fcb2.reshape(1, -1))


# speedup vs baseline: 1.0777x; 1.0108x over previous
import functools

import jax
import jax.numpy as jnp
from jax import lax
from jax.experimental import pallas as pl
from jax.experimental.pallas import tpu as pltpu
from jax.experimental.pallas import tpu_sc as plsc

N = 10000
E = 320000
D = 128
G = 128

NC = 2
NS = 16
NW = NC * NS
NPAD = 10240
ROWS = NPAD // NS
EPW = E // NW
C = 125
NCHUNK = EPW // C

RB = 2560
GRID = NPAD // RB

_MESH = dict(core_axis_name="c", subcore_axis_name="s",
             num_cores=NC, num_subcores=NS)


@functools.cache
def _make_deg_pass():
  @functools.partial(
      pl.kernel,
      out_type=jax.ShapeDtypeStruct((NC, NPAD, 16), jnp.float32),
      mesh=plsc.VectorSubcoreMesh(**_MESH),
      compiler_params=pltpu.CompilerParams(use_tc_tiling_on_sc=False),
      scratch_types=[
          pltpu.VMEM((NCHUNK, C), jnp.int32),
          pltpu.VMEM((C, 16), jnp.float32),
          pltpu.VMEM_SHARED((NPAD, 16), jnp.float32),
          pltpu.SemaphoreType.DMA,
          pltpu.SemaphoreType.DMA,
      ],
  )
  def deg_pass(dst_hbm, ones_hbm, zeros_hbm, out_hbm, dstv, ones_v, acc_sh,
               sem0, sem1):
    sems = (sem0, sem1)
    c = lax.axis_index("c")
    s = lax.axis_index("s")
    w = c * NS + s
    pltpu.async_copy(zeros_hbm.at[pl.ds(s * ROWS, ROWS), :],
                     acc_sh.at[pl.ds(s * ROWS, ROWS), :], sem0)
    pltpu.async_copy(ones_hbm, ones_v, sem1)
    pltpu.async_copy(dst_hbm.at[w], dstv, sem0)
    pltpu.make_async_copy(zeros_hbm.at[pl.ds(s * ROWS, ROWS), :],
                          acc_sh.at[pl.ds(s * ROWS, ROWS), :], sem0).wait()
    pltpu.make_async_copy(ones_hbm, ones_v, sem1).wait()
    pltpu.make_async_copy(dst_hbm.at[w], dstv, sem0).wait()
    plsc.subcore_barrier()

    def body(i, carry):
      pltpu.sync_copy(ones_v, acc_sh.at[dstv.at[i]], add=True)
      return carry

    lax.fori_loop(0, NCHUNK, body, 0)
    plsc.subcore_barrier()
    pltpu.sync_copy(acc_sh.at[pl.ds(s * ROWS, ROWS), :],
                    out_hbm.at[c, pl.ds(s * ROWS, ROWS), :])

  return deg_pass


@functools.cache
def _make_edge_pass(F):
  @functools.partial(
      pl.kernel,
      out_type=jax.ShapeDtypeStruct((NC, NPAD, F), jnp.float32),
      mesh=plsc.VectorSubcoreMesh(**_MESH),
      compiler_params=pltpu.CompilerParams(use_tc_tiling_on_sc=False),
      scratch_types=[
          pltpu.VMEM((NCHUNK, C), jnp.int32),
          pltpu.VMEM((NCHUNK, C), jnp.int32),
          pltpu.VMEM((C, F), jnp.float32),
          pltpu.VMEM((C, F), jnp.float32),
          pltpu.VMEM((C, F), jnp.float32),
          pltpu.VMEM((C, F), jnp.float32),
          pltpu.VMEM_SHARED((NPAD, F), jnp.float32),
          pltpu.SemaphoreType.DMA,
          pltpu.SemaphoreType.DMA,
          pltpu.SemaphoreType.DMA,
          pltpu.SemaphoreType.DMA,
          pltpu.SemaphoreType.DMA,
          pltpu.SemaphoreType.DMA,
          pltpu.SemaphoreType.DMA,
          pltpu.SemaphoreType.DMA,
      ],
  )
  def edge_pass(table_hbm, src_hbm, dst_hbm, zeros_hbm, out_hbm,
                srcv, dstv, m0, m1, m2, m3, acc_sh,
                g0, g1, g2, g3, s0, s1, s2, s3):
    msgs = (m0, m1, m2, m3)
    gsem = (g0, g1, g2, g3)
    ssem = (s0, s1, s2, s3)
    c = lax.axis_index("c")
    s = lax.axis_index("s")
    w = c * NS + s
    pltpu.async_copy(zeros_hbm.at[pl.ds(s * ROWS, ROWS), :],
                     acc_sh.at[pl.ds(s * ROWS, ROWS), :], ssem[0])
    pltpu.async_copy(src_hbm.at[w], srcv, ssem[1])
    pltpu.async_copy(dst_hbm.at[w], dstv, ssem[2])
    pltpu.make_async_copy(zeros_hbm.at[pl.ds(s * ROWS, ROWS), :],
                          acc_sh.at[pl.ds(s * ROWS, ROWS), :], ssem[0]).wait()
    pltpu.make_async_copy(src_hbm.at[w], srcv, ssem[1]).wait()
    pltpu.make_async_copy(dst_hbm.at[w], dstv, ssem[2]).wait()
    plsc.subcore_barrier()

    pltpu.async_copy(table_hbm.at[srcv.at[0]], msgs[0], gsem[0])
    pltpu.async_copy(table_hbm.at[srcv.at[1]], msgs[1], gsem[1])
    pltpu.async_copy(table_hbm.at[srcv.at[2]], msgs[2], gsem[2])

    def body(j, carry):
      for b in range(4):
        i = 4 * j + b
        pltpu.make_async_copy(table_hbm.at[srcv.at[i]], msgs[b],
                              gsem[b]).wait()

        @pl.when(i + 3 < NCHUNK)
        def _():
          pltpu.async_copy(table_hbm.at[srcv.at[i + 3]],
                           msgs[(b + 3) % 4], gsem[(b + 3) % 4])

        pltpu.sync_copy(msgs[b], acc_sh.at[dstv.at[i]], add=True)

      return carry

    lax.fori_loop(0, NCHUNK // 4, body, 0)
    plsc.subcore_barrier()
    pltpu.sync_copy(acc_sh.at[pl.ds(s * ROWS, ROWS), :],
                    out_hbm.at[c, pl.ds(s * ROWS, ROWS), :])

  return edge_pass


def _dinv_of(deg_ref):
  degsum = deg_ref[0] + deg_ref[1]
  return lax.rsqrt(degsum[:, 0:1] + 1.0)


def _mm0_body(x_ref, w_ref, out_ref):
  out_ref[...] = jnp.dot(x_ref[...], w_ref[...],
                         preferred_element_type=jnp.float32,
                         precision=lax.Precision.HIGHEST)


def _mm0(x, W0):
  return pl.pallas_call(
      _mm0_body,
      grid=(GRID,),
      in_specs=[
          pl.BlockSpec((RB, D), lambda i: (i, 0)),
          pl.BlockSpec((D, 32), lambda i: (0, 0)),
      ],
      out_specs=pl.BlockSpec((RB, 32), lambda i: (i, 0)),
      out_shape=jax.ShapeDtypeStruct((NPAD, 32), jnp.float32),
  )(x, W0)


def _scale_body(h_ref, deg_ref, out_ref):
  out_ref[...] = h_ref[...] * _dinv_of(deg_ref)


def _first_layer(x, W0, accdeg):
  h0 = _mm0(x, W0)
  return pl.pallas_call(
      _scale_body,
      grid=(GRID,),
      in_specs=[
          pl.BlockSpec((RB, 32), lambda i: (i, 0)),
          pl.BlockSpec((NC, RB, 16), lambda i: (0, i, 0)),
      ],
      out_specs=pl.BlockSpec((RB, 32), lambda i: (i, 0)),
      out_shape=jax.ShapeDtypeStruct((NPAD, 32), jnp.float32),
  )(h0, accdeg)


def _mid_body(acc_ref, hhat_ref, deg_ref, b_ref, w_ref, out_ref):
  dinv = _dinv_of(deg_ref)
  xn = jax.nn.relu(dinv * (acc_ref[0] + acc_ref[1] + hhat_ref[...])
                   + b_ref[...])
  h = jnp.dot(xn, w_ref[...], preferred_element_type=jnp.float32, precision=lax.Precision.HIGHEST)
  out_ref[...] = h * dinv


def _mid_layer(F, F2, acc, hhat, accdeg, b, W):
  return pl.pallas_call(
      _mid_body,
      grid=(GRID,),
      in_specs=[
          pl.BlockSpec((NC, RB, F), lambda i: (0, i, 0)),
          pl.BlockSpec((RB, F), lambda i: (i, 0)),
          pl.BlockSpec((NC, RB, 16), lambda i: (0, i, 0)),
          pl.BlockSpec((1, F), lambda i: (0, 0)),
          pl.BlockSpec((F, F2), lambda i: (0, 0)),
      ],
      out_specs=pl.BlockSpec((RB, F2), lambda i: (i, 0)),
      out_shape=jax.ShapeDtypeStruct((NPAD, F2), jnp.float32),
  )(acc, hhat, accdeg, b, W)


def _final_body(acc_ref, hhat_ref, deg_ref, b_ref, batch_ref,
                fw1_ref, fb1_ref, fw2_ref, fb2_ref, out_ref,
                sums_ref, cnt_ref):
  i = pl.program_id(0)

  @pl.when(i == 0)
  def _():
    sums_ref[...] = jnp.zeros_like(sums_ref)
    cnt_ref[...] = jnp.zeros_like(cnt_ref)

  dinv = _dinv_of(deg_ref)
  x3 = jax.nn.relu(dinv * (acc_ref[0] + acc_ref[1] + hhat_ref[...])
                   + b_ref[...])
  seg = lax.broadcasted_iota(jnp.int32, (G, 1), 0)
  onehot_t = (seg == batch_ref[...]).astype(jnp.float32)
  sums_ref[...] += jnp.dot(onehot_t, x3, preferred_element_type=jnp.float32, precision=lax.Precision.HIGHEST)
  cnt_ref[...] += jnp.sum(onehot_t, axis=1, keepdims=True)

  @pl.when(i == GRID - 1)
  def _():
    pooled = sums_ref[...] / jnp.maximum(cnt_ref[...], 1.0)
    z = jax.nn.relu(jnp.dot(pooled, fw1_ref[...],
                            preferred_element_type=jnp.float32, precision=lax.Precision.HIGHEST) + fb1_ref[...])
    z2 = jnp.dot(z, fw2_ref[...],
                 preferred_element_type=jnp.float32, precision=lax.Precision.HIGHEST) + fb2_ref[...]
    out_ref[...] = 1.0 / (1.0 + jnp.exp(-z2))


def _final_layer(acc, hhat, accdeg, b2, batch_p, fcW1, fcb1, fcW2, fcb2):
  return pl.pallas_call(
      _final_body,
      grid=(GRID,),
      in_specs=[
          pl.BlockSpec((NC, RB, 64), lambda i: (0, i, 0)),
          pl.BlockSpec((RB, 64), lambda i: (i, 0)),
          pl.BlockSpec((NC, RB, 16), lambda i: (0, i, 0)),
          pl.BlockSpec((1, 64), lambda i: (0, 0)),
          pl.BlockSpec((1, RB), lambda i: (0, i)),
          pl.BlockSpec((64, 32), lambda i: (0, 0)),
          pl.BlockSpec((1, 32), lambda i: (0, 0)),
          pl.BlockSpec((32, 1), lambda i: (0, 0)),
          pl.BlockSpec((1, 1), lambda i: (0, 0)),
      ],
      out_specs=pl.BlockSpec((G, 1), lambda i: (0, 0)),
      out_shape=jax.ShapeDtypeStruct((G, 1), jnp.float32),
      scratch_shapes=[
          pltpu.VMEM((G, 64), jnp.float32),
          pltpu.VMEM((G, 1), jnp.float32),
      ],
  )(acc, hhat, accdeg, b2, batch_p, fcW1, fcb1, fcW2, fcb2)


def kernel(x, edge_index, batch, W0, b0, W1, b1, W2, b2,
           fcW1, fcb1, fcW2, fcb2):
  xp = jnp.pad(x, ((0, NPAD - N), (0, 0)))
  batch_p = jnp.pad(batch, (0, NPAD - N), constant_values=G).reshape(1, NPAD)
  src = edge_index[0].reshape(NW, NCHUNK, C)
  dst = edge_index[1].reshape(NW, NCHUNK, C)
  ones_c16 = jnp.ones((C, 16), jnp.float32)
  z16 = jnp.zeros((NPAD, 16), jnp.float32)
  z32 = jnp.zeros((NPAD, 32), jnp.float32)
  z48 = jnp.zeros((NPAD, 48), jnp.float32)
  z64 = jnp.zeros((NPAD, 64), jnp.float32)

  accdeg = _make_deg_pass()(dst, ones_c16, z16)
  hhat0 = _first_layer(xp, W0, accdeg)
  acc0 = _make_edge_pass(32)(hhat0, src, dst, z32)
  hhat1 = _mid_layer(32, 48, acc0, hhat0, accdeg, b0.reshape(1, -1), W1)
  acc1 = _make_edge_pass(48)(hhat1, src, dst, z48)
  hhat2 = _mid_layer(48, 64, acc1, hhat1, accdeg, b1.reshape(1, -1), W2)
  acc2 = _make_edge_pass(64)(hhat2, src, dst, z64)
  return _final_layer(acc2, hhat2, accdeg, b2.reshape(1, -1), batch_p,
                      fcW1, fcb1.reshape(1, -1), fcW2, fcb2.reshape(1, -1))
